# speedup vs baseline: 2.1247x; 2.1247x over previous
import jax
import jax.numpy as jnp
from jax.experimental import pallas as pl

_EPS = 1e-12


def _body(x_ref, p_ref, g_ref, b_ref, o_ref):
    x = x_ref[0]
    p = p_ref[...]
    emb = x + p
    mean = jnp.mean(emb, axis=1, keepdims=True)
    d = emb - mean
    var = jnp.mean(d * d, axis=1, keepdims=True)
    std = jnp.sqrt(var)
    inv = jax.lax.rsqrt(std + _EPS)
    o_ref[0] = g_ref[...] * (d * inv) + b_ref[...]


def kernel(input, pos_table, gamma, beta):
    B, S, D = input.shape
    BS = 512
    grid = (S // BS, B)
    out = pl.pallas_call(
        _body,
        grid=grid,
        in_specs=[
            pl.BlockSpec((1, BS, D), lambda s, b: (b, s, 0)),
            pl.BlockSpec((BS, D), lambda s, b: (s, 0)),
            pl.BlockSpec((1, D), lambda s, b: (0, 0)),
            pl.BlockSpec((1, D), lambda s, b: (0, 0)),
        ],
        out_specs=pl.BlockSpec((1, BS, D), lambda s, b: (b, s, 0)),
        out_shape=jax.ShapeDtypeStruct((B, S, D), jnp.float32),
    )(input, pos_table, gamma.reshape(1, D), beta.reshape(1, D))
    return out


# speedup vs baseline: 2.4244x; 1.1410x over previous
import jax
import jax.numpy as jnp
from jax.experimental import pallas as pl

_EPS = 1e-12


def _body(x_ref, p_ref, g_ref, b_ref, o_ref):
    x = x_ref[0]
    p = p_ref[...]
    emb = x + p
    mean = jnp.mean(emb, axis=1, keepdims=True)
    d = emb - mean
    var = jnp.mean(d * d, axis=1, keepdims=True)
    std = jnp.sqrt(var)
    inv = jax.lax.rsqrt(std + _EPS)
    o_ref[0] = g_ref[...] * (d * inv) + b_ref[...]


def kernel(input, pos_table, gamma, beta):
    B, S, D = input.shape
    BS = 1024
    grid = (S // BS, B)
    out = pl.pallas_call(
        _body,
        grid=grid,
        in_specs=[
            pl.BlockSpec((1, BS, D), lambda s, b: (b, s, 0)),
            pl.BlockSpec((BS, D), lambda s, b: (s, 0)),
            pl.BlockSpec((1, D), lambda s, b: (0, 0)),
            pl.BlockSpec((1, D), lambda s, b: (0, 0)),
        ],
        out_specs=pl.BlockSpec((1, BS, D), lambda s, b: (b, s, 0)),
        out_shape=jax.ShapeDtypeStruct((B, S, D), jnp.float32),
    )(input, pos_table, gamma.reshape(1, D), beta.reshape(1, D))
    return out


# speedup vs baseline: 2.5595x; 1.0557x over previous
import jax
import jax.numpy as jnp
from jax.experimental import pallas as pl

_EPS = 1e-12


def _body(x_ref, p_ref, g_ref, b_ref, o_ref):
    x = x_ref[0]
    p = p_ref[...]
    emb = x + p
    mean = jnp.mean(emb, axis=1, keepdims=True)
    d = emb - mean
    var = jnp.mean(d * d, axis=1, keepdims=True)
    std = jnp.sqrt(var)
    inv = jax.lax.rsqrt(std + _EPS)
    o_ref[0] = g_ref[...] * (d * inv) + b_ref[...]


def kernel(input, pos_table, gamma, beta):
    B, S, D = input.shape
    BS = 2048
    grid = (S // BS, B)
    out = pl.pallas_call(
        _body,
        grid=grid,
        in_specs=[
            pl.BlockSpec((1, BS, D), lambda s, b: (b, s, 0)),
            pl.BlockSpec((BS, D), lambda s, b: (s, 0)),
            pl.BlockSpec((1, D), lambda s, b: (0, 0)),
            pl.BlockSpec((1, D), lambda s, b: (0, 0)),
        ],
        out_specs=pl.BlockSpec((1, BS, D), lambda s, b: (b, s, 0)),
        out_shape=jax.ShapeDtypeStruct((B, S, D), jnp.float32),
    )(input, pos_table, gamma.reshape(1, D), beta.reshape(1, D))
    return out


# speedup vs baseline: 2.7226x; 1.0637x over previous
import jax
import jax.numpy as jnp
from jax.experimental import pallas as pl

_EPS = 1e-12


def _body(x_ref, p_ref, g_ref, b_ref, o_ref):
    x = x_ref[...]
    p = p_ref[...]
    emb = x + p[None]
    mean = jnp.mean(emb, axis=2, keepdims=True)
    d = emb - mean
    var = jnp.mean(d * d, axis=2, keepdims=True)
    std = jnp.sqrt(var)
    inv = jax.lax.rsqrt(std + _EPS)
    o_ref[...] = g_ref[...] * (d * inv) + b_ref[...]


def kernel(input, pos_table, gamma, beta):
    B, S, D = input.shape
    BS = 512
    grid = (S // BS,)
    out = pl.pallas_call(
        _body,
        grid=grid,
        in_specs=[
            pl.BlockSpec((B, BS, D), lambda s: (0, s, 0)),
            pl.BlockSpec((BS, D), lambda s: (s, 0)),
            pl.BlockSpec((1, D), lambda s: (0, 0)),
            pl.BlockSpec((1, D), lambda s: (0, 0)),
        ],
        out_specs=pl.BlockSpec((B, BS, D), lambda s: (0, s, 0)),
        out_shape=jax.ShapeDtypeStruct((B, S, D), jnp.float32),
    )(input, pos_table, gamma.reshape(1, D), beta.reshape(1, D))
    return out
